# Initial kernel scaffold; baseline (speedup 1.0000x reference)
import jax
import jax.numpy as jnp
from jax.experimental import pallas as pl


def kernel(x, edge_index, W_enc, b_enc, Wx1, bx1, Wx2, bx2, Wh1, bh1, Wh2, bh2):
    raise NotImplementedError("write your pallas kernel here")


# speedup vs baseline: 6.9370x; 6.9370x over previous
import functools

import jax
import jax.numpy as jnp
from jax import lax
from jax.experimental import pallas as pl
from jax.experimental.pallas import tpu as pltpu
from jax.experimental.pallas import tpu_sc as plsc

N = 10000
NPAD = 10240
E = 320000
F = 128
CHUNK = 128
EPAD = 2560 * CHUNK
CPW = 2560 // 32
ROWS_PER_TILE = NPAD // 16

_f32 = jnp.float32


def _encode_body(x_ref, w_ref, out_ref):
    out_ref[...] = jnp.dot(x_ref[...], w_ref[...], preferred_element_type=_f32)


def _encode(x, w, blk=1280):
    grid = NPAD // blk
    return pl.pallas_call(
        _encode_body,
        grid=(grid,),
        in_specs=[
            pl.BlockSpec((blk, F), lambda i: (i, 0)),
            pl.BlockSpec((F, F), lambda i: (0, 0)),
        ],
        out_specs=pl.BlockSpec((blk, F), lambda i: (i, 0)),
        out_shape=jax.ShapeDtypeStruct((NPAD, F), _f32),
    )(x, w)


def _finish_body(parts_ref, cnts_ref, benc_ref, out_ref, *, with_enc):
    tot = parts_ref[0] + parts_ref[1]
    cnt = (cnts_ref[0] + cnts_ref[1])[:, 0:1]
    inv = 1.0 / jnp.maximum(cnt, 1.0)
    if with_enc:
        out_ref[...] = jnp.maximum(tot * inv + benc_ref[...], 0.0)
    else:
        out_ref[...] = tot * inv


def _finish(parts, cnts, benc, with_enc, blk=1280):
    grid = NPAD // blk
    body = functools.partial(_finish_body, with_enc=with_enc)
    return pl.pallas_call(
        body,
        grid=(grid,),
        in_specs=[
            pl.BlockSpec((2, blk, F), lambda i: (0, i, 0)),
            pl.BlockSpec((2, blk, F), lambda i: (0, i, 0)),
            pl.BlockSpec((1, F), lambda i: (0, 0)),
        ],
        out_specs=pl.BlockSpec((blk, F), lambda i: (i, 0)),
        out_shape=jax.ShapeDtypeStruct((NPAD, F), _f32),
    )(parts, cnts, benc)


def _mlp_body(in_ref, w1_ref, b1_ref, w2_ref, b2_ref, out_ref):
    t = jnp.dot(in_ref[...], w1_ref[...], preferred_element_type=_f32)
    t = jnp.maximum(t + b1_ref[...], 0.0)
    out_ref[...] = (
        jnp.dot(t, w2_ref[...], preferred_element_type=_f32) + b2_ref[...]
    )


def _mlp(inp, w1, b1, w2, b2, blk=1280):
    grid = NPAD // blk
    return pl.pallas_call(
        _mlp_body,
        grid=(grid,),
        in_specs=[
            pl.BlockSpec((blk, F), lambda i: (i, 0)),
            pl.BlockSpec((F, F), lambda i: (0, 0)),
            pl.BlockSpec((1, F), lambda i: (0, 0)),
            pl.BlockSpec((F, F), lambda i: (0, 0)),
            pl.BlockSpec((1, F), lambda i: (0, 0)),
        ],
        out_specs=pl.BlockSpec((blk, F), lambda i: (i, 0)),
        out_shape=jax.ShapeDtypeStruct((NPAD, F), _f32),
    )(inp, w1, b1, w2, b2)


def _zero_rows(zbuf, acc, row0, width):
    z = jnp.zeros((16,), _f32)
    for i in range(16):
        for k in range(width // 16):
            zbuf[i, pl.ds(k * 16, 16)] = z

    def zero_body(i, carry):
        pltpu.sync_copy(zbuf, acc.at[pl.ds(row0 + i * 16, 16)])
        return carry

    lax.fori_loop(0, ROWS_PER_TILE // 16, zero_body, 0)


def _agg_body(table, src2d, dst2d, out, src_v, dst_v, rows_v, zbuf, acc, sem):
    c = lax.axis_index("c")
    s = lax.axis_index("s")
    wid = c * 16 + s
    row0 = s * ROWS_PER_TILE

    _zero_rows(zbuf, acc, row0, F)
    plsc.subcore_barrier()

    pltpu.sync_copy(src2d.at[pl.ds(wid * CPW, CPW)], src_v)
    pltpu.sync_copy(dst2d.at[pl.ds(wid * CPW, CPW)], dst_v)

    def chunk_body(j, carry):
        pltpu.async_copy(table.at[src_v.at[j]], rows_v, sem).wait()
        pltpu.sync_copy(rows_v, acc.at[dst_v.at[j]], add=True)
        return carry

    lax.fori_loop(0, CPW, chunk_body, 0)
    plsc.subcore_barrier()

    pltpu.sync_copy(
        acc.at[pl.ds(row0, ROWS_PER_TILE)],
        out.at[c, pl.ds(row0, ROWS_PER_TILE)],
    )


def _aggregate(table, src2d, dst2d):
    mesh = plsc.VectorSubcoreMesh(core_axis_name="c", subcore_axis_name="s")
    kern = functools.partial(
        pl.kernel,
        mesh=mesh,
        out_type=jax.ShapeDtypeStruct((2, NPAD, F), _f32),
        scratch_types=[
            pltpu.VMEM((CPW, CHUNK), jnp.int32),
            pltpu.VMEM((CPW, CHUNK), jnp.int32),
            pltpu.VMEM((CHUNK, F), _f32),
            pltpu.VMEM((16, F), _f32),
            pltpu.VMEM_SHARED((NPAD, F), _f32),
            pltpu.SemaphoreType.DMA,
        ],
    )(_agg_body)
    return kern(table, src2d, dst2d)


def _count_body(dst2d, out_cnt, dst_v, ones_v, zbuf, cacc):
    c = lax.axis_index("c")
    s = lax.axis_index("s")
    wid = c * 16 + s
    row0 = s * ROWS_PER_TILE

    _zero_rows(zbuf, cacc, row0, F)
    onehot = jnp.where(lax.iota(jnp.int32, 16) == 0, 1.0, 0.0).astype(_f32)
    z = jnp.zeros((16,), _f32)
    for i in range(CHUNK):
        ones_v[i, pl.ds(0, 16)] = onehot
        for k in range(1, F // 16):
            ones_v[i, pl.ds(k * 16, 16)] = z
    plsc.subcore_barrier()

    pltpu.sync_copy(dst2d.at[pl.ds(wid * CPW, CPW)], dst_v)

    def chunk_body(j, carry):
        pltpu.sync_copy(ones_v, cacc.at[dst_v.at[j]], add=True)
        return carry

    lax.fori_loop(0, CPW, chunk_body, 0)
    plsc.subcore_barrier()

    pltpu.sync_copy(
        cacc.at[pl.ds(row0, ROWS_PER_TILE)],
        out_cnt.at[c, pl.ds(row0, ROWS_PER_TILE)],
    )


def _count(dst2d):
    mesh = plsc.VectorSubcoreMesh(core_axis_name="c", subcore_axis_name="s")
    kern = functools.partial(
        pl.kernel,
        mesh=mesh,
        out_type=jax.ShapeDtypeStruct((2, NPAD, F), _f32),
        scratch_types=[
            pltpu.VMEM((CPW, CHUNK), jnp.int32),
            pltpu.VMEM((CHUNK, F), _f32),
            pltpu.VMEM((16, F), _f32),
            pltpu.VMEM_SHARED((NPAD, F), _f32),
        ],
    )(_count_body)
    return kern(dst2d)


def kernel(x, edge_index, W_enc, b_enc, Wx1, bx1, Wx2, bx2, Wh1, bh1, Wh2, bh2):
    xpad = jnp.concatenate([x, jnp.zeros((NPAD - N, F), _f32)], axis=0)
    src = edge_index[0]
    dst = edge_index[1]
    pad = EPAD - E
    pad_ids = lax.iota(jnp.int32, pad)
    srcp = jnp.concatenate([src, pad_ids % N]).reshape(-1, CHUNK)
    dstp = jnp.concatenate([dst, N + pad_ids % (NPAD - N)]).reshape(-1, CHUNK)

    table1 = _encode(xpad, W_enc)
    cnts = _count(dstp)
    parts1 = _aggregate(table1, srcp, dstp)
    h_full = _finish(parts1, cnts, b_enc.reshape(1, F), with_enc=True)
    parts2 = _aggregate(h_full, srcp, dstp)

    x_hat = _mlp(h_full, Wx1, bx1.reshape(1, F), Wx2, bx2.reshape(1, F))
    h_bar = _finish(parts2, cnts, b_enc.reshape(1, F), with_enc=False)
    m_hat = _mlp(h_bar, Wh1, bh1.reshape(1, F), Wh2, bh2.reshape(1, F))

    return (h_full[:N], x_hat[:N], m_hat[:N])


# speedup vs baseline: 9.3050x; 1.3414x over previous
import functools

import jax
import jax.numpy as jnp
from jax import lax
from jax.experimental import pallas as pl
from jax.experimental.pallas import tpu as pltpu
from jax.experimental.pallas import tpu_sc as plsc

N = 10000
NPAD = 10240
E = 320000
F = 128
CHUNK = 128
EPAD = 2560 * CHUNK
CPW = 2560 // 32
ROWS_PER_TILE = NPAD // 16
HALF = CPW // 2

_f32 = jnp.float32


def _encode_body(x_ref, w_ref, out_ref):
    out_ref[...] = jnp.dot(x_ref[...], w_ref[...], preferred_element_type=_f32)


def _encode(x, w, blk=1280):
    grid = NPAD // blk
    return pl.pallas_call(
        _encode_body,
        grid=(grid,),
        in_specs=[
            pl.BlockSpec((blk, F), lambda i: (i, 0)),
            pl.BlockSpec((F, F), lambda i: (0, 0)),
        ],
        out_specs=pl.BlockSpec((blk, F), lambda i: (i, 0)),
        out_shape=jax.ShapeDtypeStruct((NPAD, F), _f32),
    )(x, w)


def _finish_body(parts_ref, cnts_ref, benc_ref, out_ref, *, with_enc):
    tot = parts_ref[0] + parts_ref[1]
    cnt = (cnts_ref[0] + cnts_ref[1])[:, 0:1]
    inv = 1.0 / jnp.maximum(cnt, 1.0)
    if with_enc:
        out_ref[...] = jnp.maximum(tot * inv + benc_ref[...], 0.0)
    else:
        out_ref[...] = tot * inv


def _finish(parts, cnts, benc, with_enc, blk=1280):
    grid = NPAD // blk
    body = functools.partial(_finish_body, with_enc=with_enc)
    return pl.pallas_call(
        body,
        grid=(grid,),
        in_specs=[
            pl.BlockSpec((2, blk, F), lambda i: (0, i, 0)),
            pl.BlockSpec((2, blk, F), lambda i: (0, i, 0)),
            pl.BlockSpec((1, F), lambda i: (0, 0)),
        ],
        out_specs=pl.BlockSpec((blk, F), lambda i: (i, 0)),
        out_shape=jax.ShapeDtypeStruct((NPAD, F), _f32),
    )(parts, cnts, benc)


def _mlp_body(in_ref, w1_ref, b1_ref, w2_ref, b2_ref, out_ref):
    t = jnp.dot(in_ref[...], w1_ref[...], preferred_element_type=_f32)
    t = jnp.maximum(t + b1_ref[...], 0.0)
    out_ref[...] = (
        jnp.dot(t, w2_ref[...], preferred_element_type=_f32) + b2_ref[...]
    )


def _mlp(inp, w1, b1, w2, b2, blk=1280):
    grid = NPAD // blk
    return pl.pallas_call(
        _mlp_body,
        grid=(grid,),
        in_specs=[
            pl.BlockSpec((blk, F), lambda i: (i, 0)),
            pl.BlockSpec((F, F), lambda i: (0, 0)),
            pl.BlockSpec((1, F), lambda i: (0, 0)),
            pl.BlockSpec((F, F), lambda i: (0, 0)),
            pl.BlockSpec((1, F), lambda i: (0, 0)),
        ],
        out_specs=pl.BlockSpec((blk, F), lambda i: (i, 0)),
        out_shape=jax.ShapeDtypeStruct((NPAD, F), _f32),
    )(inp, w1, b1, w2, b2)


def _zero_rows(zbuf, acc, row0, width):
    z = jnp.zeros((16,), _f32)
    for i in range(16):
        for k in range(width // 16):
            zbuf[i, pl.ds(k * 16, 16)] = z

    def zero_body(i, carry):
        pltpu.sync_copy(zbuf, acc.at[pl.ds(row0 + i * 16, 16)])
        return carry

    lax.fori_loop(0, ROWS_PER_TILE // 16, zero_body, 0)


def _agg_body(table, src2d, dst2d, out, src_v, dst_v, rows_a, rows_b, zbuf,
              acc, sem_a, sem_b):
    c = lax.axis_index("c")
    s = lax.axis_index("s")
    wid = c * 16 + s
    row0 = s * ROWS_PER_TILE

    _zero_rows(zbuf, acc, row0, F)
    plsc.subcore_barrier()

    for half in range(2):
        base = wid * CPW + half * HALF
        pltpu.sync_copy(src2d.at[pl.ds(base, HALF)], src_v)
        pltpu.sync_copy(dst2d.at[pl.ds(base, HALF)], dst_v)
        pltpu.async_copy(table.at[src_v.at[0]], rows_a, sem_a)

        def chunk_body(j, carry):
            jj = 2 * j
            pltpu.async_copy(table.at[src_v.at[jj + 1]], rows_b, sem_b)
            pltpu.make_async_copy(table.at[src_v.at[jj]], rows_a, sem_a).wait()
            pltpu.sync_copy(rows_a, acc.at[dst_v.at[jj]], add=True)
            nxt = jnp.minimum(jj + 2, HALF - 1)
            pltpu.async_copy(table.at[src_v.at[nxt]], rows_a, sem_a)
            pltpu.make_async_copy(table.at[src_v.at[jj]], rows_b, sem_b).wait()
            pltpu.sync_copy(rows_b, acc.at[dst_v.at[jj + 1]], add=True)
            return carry

        lax.fori_loop(0, HALF // 2, chunk_body, 0)
        pltpu.make_async_copy(table.at[src_v.at[0]], rows_a, sem_a).wait()
    plsc.subcore_barrier()

    pltpu.sync_copy(
        acc.at[pl.ds(row0, ROWS_PER_TILE)],
        out.at[c, pl.ds(row0, ROWS_PER_TILE)],
    )


def _aggregate(table, src2d, dst2d):
    mesh = plsc.VectorSubcoreMesh(core_axis_name="c", subcore_axis_name="s")
    kern = functools.partial(
        pl.kernel,
        mesh=mesh,
        out_type=jax.ShapeDtypeStruct((2, NPAD, F), _f32),
        scratch_types=[
            pltpu.VMEM((HALF, CHUNK), jnp.int32),
            pltpu.VMEM((HALF, CHUNK), jnp.int32),
            pltpu.VMEM((CHUNK, F), _f32),
            pltpu.VMEM((CHUNK, F), _f32),
            pltpu.VMEM((16, F), _f32),
            pltpu.VMEM_SHARED((NPAD, F), _f32),
            pltpu.SemaphoreType.DMA,
            pltpu.SemaphoreType.DMA,
        ],
    )(_agg_body)
    return kern(table, src2d, dst2d)


def _count_body(dst2d, out_cnt, dst_v, ones_v, zbuf, cacc):
    c = lax.axis_index("c")
    s = lax.axis_index("s")
    wid = c * 16 + s
    row0 = s * ROWS_PER_TILE

    _zero_rows(zbuf, cacc, row0, F)
    onehot = jnp.where(lax.iota(jnp.int32, 16) == 0, 1.0, 0.0).astype(_f32)
    z = jnp.zeros((16,), _f32)
    for i in range(CHUNK):
        ones_v[i, pl.ds(0, 16)] = onehot
        for k in range(1, F // 16):
            ones_v[i, pl.ds(k * 16, 16)] = z
    plsc.subcore_barrier()

    pltpu.sync_copy(dst2d.at[pl.ds(wid * CPW, CPW)], dst_v)

    def chunk_body(j, carry):
        pltpu.sync_copy(ones_v, cacc.at[dst_v.at[j]], add=True)
        return carry

    lax.fori_loop(0, CPW, chunk_body, 0)
    plsc.subcore_barrier()

    pltpu.sync_copy(
        cacc.at[pl.ds(row0, ROWS_PER_TILE)],
        out_cnt.at[c, pl.ds(row0, ROWS_PER_TILE)],
    )


def _count(dst2d):
    mesh = plsc.VectorSubcoreMesh(core_axis_name="c", subcore_axis_name="s")
    kern = functools.partial(
        pl.kernel,
        mesh=mesh,
        out_type=jax.ShapeDtypeStruct((2, NPAD, F), _f32),
        scratch_types=[
            pltpu.VMEM((CPW, CHUNK), jnp.int32),
            pltpu.VMEM((CHUNK, F), _f32),
            pltpu.VMEM((16, F), _f32),
            pltpu.VMEM_SHARED((NPAD, F), _f32),
        ],
    )(_count_body)
    return kern(dst2d)


def kernel(x, edge_index, W_enc, b_enc, Wx1, bx1, Wx2, bx2, Wh1, bh1, Wh2, bh2):
    xpad = jnp.concatenate([x, jnp.zeros((NPAD - N, F), _f32)], axis=0)
    src = edge_index[0]
    dst = edge_index[1]
    pad = EPAD - E
    pad_ids = lax.iota(jnp.int32, pad)
    srcp = jnp.concatenate([src, pad_ids % N]).reshape(-1, CHUNK)
    dstp = jnp.concatenate([dst, N + pad_ids % (NPAD - N)]).reshape(-1, CHUNK)

    table1 = _encode(xpad, W_enc)
    cnts = _count(dstp)
    parts1 = _aggregate(table1, srcp, dstp)
    h_full = _finish(parts1, cnts, b_enc.reshape(1, F), with_enc=True)
    parts2 = _aggregate(h_full, srcp, dstp)

    x_hat = _mlp(h_full, Wx1, bx1.reshape(1, F), Wx2, bx2.reshape(1, F))
    h_bar = _finish(parts2, cnts, b_enc.reshape(1, F), with_enc=False)
    m_hat = _mlp(h_bar, Wh1, bh1.reshape(1, F), Wh2, bh2.reshape(1, F))

    return (h_full[:N], x_hat[:N], m_hat[:N])


# speedup vs baseline: 11.4653x; 1.2322x over previous
import functools

import jax
import jax.numpy as jnp
from jax import lax
from jax.experimental import pallas as pl
from jax.experimental.pallas import tpu as pltpu
from jax.experimental.pallas import tpu_sc as plsc

N = 10000
NPAD = 10240
E = 320000
F = 128
CW = 16
CHUNK = 128
EPAD = 2560 * CHUNK
CPW = 2560 // 32
HALF = CPW // 2
ROWS_PER_TILE = NPAD // 16

_f32 = jnp.float32


def _finish_enc_body(parts_ref, cnts_ref, wenc_ref, benc_ref,
                     w1_ref, b1_ref, w2_ref, b2_ref, h_ref, xhat_ref):
    tot = parts_ref[0] + parts_ref[1]
    cnt = (cnts_ref[0] + cnts_ref[1])[:, 0:1]
    xm = tot * (1.0 / jnp.maximum(cnt, 1.0))
    h = jnp.dot(xm, wenc_ref[...], preferred_element_type=_f32)
    h = jnp.maximum(h + benc_ref[...], 0.0)
    h_ref[...] = h
    t = jnp.dot(h, w1_ref[...], preferred_element_type=_f32)
    t = jnp.maximum(t + b1_ref[...], 0.0)
    xhat_ref[...] = (
        jnp.dot(t, w2_ref[...], preferred_element_type=_f32) + b2_ref[...]
    )


def _finish_enc(parts, cnts, wenc, benc, w1, b1, w2, b2, blk=1280):
    grid = NPAD // blk
    mat = pl.BlockSpec((F, F), lambda i: (0, 0))
    vec = pl.BlockSpec((1, F), lambda i: (0, 0))
    return pl.pallas_call(
        _finish_enc_body,
        grid=(grid,),
        in_specs=[
            pl.BlockSpec((2, blk, F), lambda i: (0, i, 0)),
            pl.BlockSpec((2, blk, CW), lambda i: (0, i, 0)),
            mat, vec, mat, vec, mat, vec,
        ],
        out_specs=(
            pl.BlockSpec((blk, F), lambda i: (i, 0)),
            pl.BlockSpec((blk, F), lambda i: (i, 0)),
        ),
        out_shape=(
            jax.ShapeDtypeStruct((NPAD, F), _f32),
            jax.ShapeDtypeStruct((NPAD, F), _f32),
        ),
    )(parts, cnts, wenc, benc, w1, b1, w2, b2)


def _finish_mlp_body(parts_ref, cnts_ref, w1_ref, b1_ref, w2_ref, b2_ref,
                     out_ref):
    tot = parts_ref[0] + parts_ref[1]
    cnt = (cnts_ref[0] + cnts_ref[1])[:, 0:1]
    hb = tot * (1.0 / jnp.maximum(cnt, 1.0))
    t = jnp.dot(hb, w1_ref[...], preferred_element_type=_f32)
    t = jnp.maximum(t + b1_ref[...], 0.0)
    out_ref[...] = (
        jnp.dot(t, w2_ref[...], preferred_element_type=_f32) + b2_ref[...]
    )


def _finish_mlp(parts, cnts, w1, b1, w2, b2, blk=1280):
    grid = NPAD // blk
    mat = pl.BlockSpec((F, F), lambda i: (0, 0))
    vec = pl.BlockSpec((1, F), lambda i: (0, 0))
    return pl.pallas_call(
        _finish_mlp_body,
        grid=(grid,),
        in_specs=[
            pl.BlockSpec((2, blk, F), lambda i: (0, i, 0)),
            pl.BlockSpec((2, blk, CW), lambda i: (0, i, 0)),
            mat, vec, mat, vec,
        ],
        out_specs=pl.BlockSpec((blk, F), lambda i: (i, 0)),
        out_shape=jax.ShapeDtypeStruct((NPAD, F), _f32),
    )(parts, cnts, w1, b1, w2, b2)


def _zero_rows(zbuf, acc, row0, width):
    z = jnp.zeros((16,), _f32)
    for i in range(16):
        for k in range(width // 16):
            zbuf[i, pl.ds(k * 16, 16)] = z

    def zero_body(i, carry):
        pltpu.sync_copy(zbuf, acc.at[pl.ds(row0 + i * 16, 16)])
        return carry

    lax.fori_loop(0, ROWS_PER_TILE // 16, zero_body, 0)


def _agg_body(table, src2d, dst2d, out, src_v, dst_v, rows_a, rows_b, zbuf,
              acc, sem_a, sem_b):
    c = lax.axis_index("c")
    s = lax.axis_index("s")
    wid = c * 16 + s
    row0 = s * ROWS_PER_TILE

    _zero_rows(zbuf, acc, row0, F)
    plsc.subcore_barrier()

    for half in range(2):
        base = wid * CPW + half * HALF
        pltpu.sync_copy(src2d.at[pl.ds(base, HALF)], src_v)
        pltpu.sync_copy(dst2d.at[pl.ds(base, HALF)], dst_v)
        pltpu.async_copy(table.at[src_v.at[0]], rows_a, sem_a)

        def chunk_body(j, carry):
            jj = 2 * j
            pltpu.async_copy(table.at[src_v.at[jj + 1]], rows_b, sem_b)
            pltpu.make_async_copy(table.at[src_v.at[jj]], rows_a, sem_a).wait()
            pltpu.sync_copy(rows_a, acc.at[dst_v.at[jj]], add=True)
            nxt = jnp.minimum(jj + 2, HALF - 1)
            pltpu.async_copy(table.at[src_v.at[nxt]], rows_a, sem_a)
            pltpu.make_async_copy(table.at[src_v.at[jj]], rows_b, sem_b).wait()
            pltpu.sync_copy(rows_b, acc.at[dst_v.at[jj + 1]], add=True)
            return carry

        lax.fori_loop(0, HALF // 2, chunk_body, 0)
        pltpu.make_async_copy(table.at[src_v.at[0]], rows_a, sem_a).wait()
    plsc.subcore_barrier()

    pltpu.sync_copy(
        acc.at[pl.ds(row0, ROWS_PER_TILE)],
        out.at[c, pl.ds(row0, ROWS_PER_TILE)],
    )


def _aggregate(table, src2d, dst2d):
    mesh = plsc.VectorSubcoreMesh(core_axis_name="c", subcore_axis_name="s")
    kern = functools.partial(
        pl.kernel,
        mesh=mesh,
        out_type=jax.ShapeDtypeStruct((2, NPAD, F), _f32),
        scratch_types=[
            pltpu.VMEM((HALF, CHUNK), jnp.int32),
            pltpu.VMEM((HALF, CHUNK), jnp.int32),
            pltpu.VMEM((CHUNK, F), _f32),
            pltpu.VMEM((CHUNK, F), _f32),
            pltpu.VMEM((16, F), _f32),
            pltpu.VMEM_SHARED((NPAD, F), _f32),
            pltpu.SemaphoreType.DMA,
            pltpu.SemaphoreType.DMA,
        ],
    )(_agg_body)
    return kern(table, src2d, dst2d)


def _count_body(dst2d, out_cnt, dst_v, ones_v, zbuf, cacc):
    c = lax.axis_index("c")
    s = lax.axis_index("s")
    wid = c * 16 + s
    row0 = s * ROWS_PER_TILE

    z = jnp.zeros((16,), _f32)
    for i in range(16):
        zbuf[i, pl.ds(0, CW)] = z

    def zero_body(i, carry):
        pltpu.sync_copy(zbuf, cacc.at[pl.ds(row0 + i * 16, 16)])
        return carry

    lax.fori_loop(0, ROWS_PER_TILE // 16, zero_body, 0)

    onehot = jnp.where(lax.iota(jnp.int32, 16) == 0, 1.0, 0.0).astype(_f32)
    for i in range(CHUNK):
        ones_v[i, pl.ds(0, CW)] = onehot
    plsc.subcore_barrier()

    pltpu.sync_copy(dst2d.at[pl.ds(wid * CPW, CPW)], dst_v)

    def chunk_body(j, carry):
        pltpu.sync_copy(ones_v, cacc.at[dst_v.at[j]], add=True)
        return carry

    lax.fori_loop(0, CPW, chunk_body, 0)
    plsc.subcore_barrier()

    pltpu.sync_copy(
        cacc.at[pl.ds(row0, ROWS_PER_TILE)],
        out_cnt.at[c, pl.ds(row0, ROWS_PER_TILE)],
    )


def _count(dst2d):
    mesh = plsc.VectorSubcoreMesh(core_axis_name="c", subcore_axis_name="s")
    kern = functools.partial(
        pl.kernel,
        mesh=mesh,
        out_type=jax.ShapeDtypeStruct((2, NPAD, CW), _f32),
        scratch_types=[
            pltpu.VMEM((CPW, CHUNK), jnp.int32),
            pltpu.VMEM((CHUNK, CW), _f32),
            pltpu.VMEM((16, CW), _f32),
            pltpu.VMEM_SHARED((NPAD, CW), _f32),
        ],
        compiler_params=pltpu.CompilerParams(use_tc_tiling_on_sc=False),
    )(_count_body)
    return kern(dst2d)


def kernel(x, edge_index, W_enc, b_enc, Wx1, bx1, Wx2, bx2, Wh1, bh1, Wh2, bh2):
    xpad = jnp.concatenate([x, jnp.zeros((NPAD - N, F), _f32)], axis=0)
    src = edge_index[0]
    dst = edge_index[1]
    pad = EPAD - E
    pad_ids = lax.iota(jnp.int32, pad)
    srcp = jnp.concatenate([src, pad_ids % N]).reshape(-1, CHUNK)
    dstp = jnp.concatenate([dst, N + pad_ids % (NPAD - N)]).reshape(-1, CHUNK)

    cnts = _count(dstp)
    parts1 = _aggregate(xpad, srcp, dstp)
    h_full, x_hat = _finish_enc(
        parts1, cnts, W_enc, b_enc.reshape(1, F),
        Wx1, bx1.reshape(1, F), Wx2, bx2.reshape(1, F),
    )
    parts2 = _aggregate(h_full, srcp, dstp)
    m_hat = _finish_mlp(
        parts2, cnts, Wh1, bh1.reshape(1, F), Wh2, bh2.reshape(1, F),
    )

    return (h_full[:N], x_hat[:N], m_hat[:N])


# speedup vs baseline: 11.6718x; 1.0180x over previous
import functools

import jax
import jax.numpy as jnp
from jax import lax
from jax.experimental import pallas as pl
from jax.experimental.pallas import tpu as pltpu
from jax.experimental.pallas import tpu_sc as plsc

N = 10000
E = 320000
F = 128
CW = 16
CHUNK = 125
CPW = E // 32 // CHUNK
HALF = CPW // 2
ROWS_PER_TILE = N // 16
OWN = 632
OWN_LAST = N - 15 * OWN

_f32 = jnp.float32


def _finish_enc_body(parts_ref, cnts_ref, wenc_ref, benc_ref,
                     w1_ref, b1_ref, w2_ref, b2_ref, h_ref, xhat_ref):
    tot = parts_ref[0] + parts_ref[1]
    cnt = (cnts_ref[0] + cnts_ref[1])[:, 0:1]
    xm = tot * (1.0 / jnp.maximum(cnt, 1.0))
    h = jnp.dot(xm, wenc_ref[...], preferred_element_type=_f32)
    h = jnp.maximum(h + benc_ref[...], 0.0)
    h_ref[...] = h
    t = jnp.dot(h, w1_ref[...], preferred_element_type=_f32)
    t = jnp.maximum(t + b1_ref[...], 0.0)
    xhat_ref[...] = (
        jnp.dot(t, w2_ref[...], preferred_element_type=_f32) + b2_ref[...]
    )


def _finish_enc(parts, cnts, wenc, benc, w1, b1, w2, b2, blk=1000):
    grid = N // blk
    mat = pl.BlockSpec((F, F), lambda i: (0, 0))
    vec = pl.BlockSpec((1, F), lambda i: (0, 0))
    return pl.pallas_call(
        _finish_enc_body,
        grid=(grid,),
        in_specs=[
            pl.BlockSpec((2, blk, F), lambda i: (0, i, 0)),
            pl.BlockSpec((2, blk, CW), lambda i: (0, i, 0)),
            mat, vec, mat, vec, mat, vec,
        ],
        out_specs=(
            pl.BlockSpec((blk, F), lambda i: (i, 0)),
            pl.BlockSpec((blk, F), lambda i: (i, 0)),
        ),
        out_shape=(
            jax.ShapeDtypeStruct((N, F), _f32),
            jax.ShapeDtypeStruct((N, F), _f32),
        ),
    )(parts, cnts, wenc, benc, w1, b1, w2, b2)


def _finish_mlp_body(parts_ref, cnts_ref, w1_ref, b1_ref, w2_ref, b2_ref,
                     out_ref):
    tot = parts_ref[0] + parts_ref[1]
    cnt = (cnts_ref[0] + cnts_ref[1])[:, 0:1]
    hb = tot * (1.0 / jnp.maximum(cnt, 1.0))
    t = jnp.dot(hb, w1_ref[...], preferred_element_type=_f32)
    t = jnp.maximum(t + b1_ref[...], 0.0)
    out_ref[...] = (
        jnp.dot(t, w2_ref[...], preferred_element_type=_f32) + b2_ref[...]
    )


def _finish_mlp(parts, cnts, w1, b1, w2, b2, blk=1000):
    grid = N // blk
    mat = pl.BlockSpec((F, F), lambda i: (0, 0))
    vec = pl.BlockSpec((1, F), lambda i: (0, 0))
    return pl.pallas_call(
        _finish_mlp_body,
        grid=(grid,),
        in_specs=[
            pl.BlockSpec((2, blk, F), lambda i: (0, i, 0)),
            pl.BlockSpec((2, blk, CW), lambda i: (0, i, 0)),
            mat, vec, mat, vec,
        ],
        out_specs=pl.BlockSpec((blk, F), lambda i: (i, 0)),
        out_shape=jax.ShapeDtypeStruct((N, F), _f32),
    )(parts, cnts, w1, b1, w2, b2)


def _fill_zbuf(zbuf, width):
    z = jnp.zeros((16,), _f32)
    for i in range(16):
        for k in range(width // 16):
            zbuf[i, pl.ds(k * 16, 16)] = z


def _zero_rows(zbuf, acc, row0, nrows16, tail):
    def zero_body(i, carry):
        pltpu.sync_copy(zbuf, acc.at[pl.ds(row0 + i * 16, 16)])
        return carry

    lax.fori_loop(0, nrows16, zero_body, 0)
    if tail:
        pltpu.sync_copy(
            zbuf.at[pl.ds(0, tail)],
            acc.at[pl.ds(row0 + nrows16 * 16, tail)],
        )


def _agg_body(table, src2d, dst2d, out, src_v, dst_v, rows_a, rows_b, zbuf,
              acc, sem_a, sem_b):
    c = lax.axis_index("c")
    s = lax.axis_index("s")
    wid = c * 16 + s
    row0 = s * OWN
    _fill_zbuf(zbuf, F)
    nz16 = jnp.where(s < 15, OWN // 16, OWN_LAST // 16)
    _zero_rows(zbuf, acc, row0, nz16, 8)
    plsc.subcore_barrier()

    for half in range(2):
        base = wid * CPW + half * HALF
        pltpu.sync_copy(src2d.at[pl.ds(base, HALF)], src_v)
        pltpu.sync_copy(dst2d.at[pl.ds(base, HALF)], dst_v)
        pltpu.async_copy(table.at[src_v.at[0]], rows_a, sem_a)

        def chunk_body(j, carry):
            jj = 2 * j
            pltpu.async_copy(table.at[src_v.at[jj + 1]], rows_b, sem_b)
            pltpu.make_async_copy(table.at[src_v.at[jj]], rows_a, sem_a).wait()
            pltpu.sync_copy(rows_a, acc.at[dst_v.at[jj]], add=True)
            nxt = jnp.minimum(jj + 2, HALF - 1)
            pltpu.async_copy(table.at[src_v.at[nxt]], rows_a, sem_a)
            pltpu.make_async_copy(table.at[src_v.at[jj]], rows_b, sem_b).wait()
            pltpu.sync_copy(rows_b, acc.at[dst_v.at[jj + 1]], add=True)
            return carry

        lax.fori_loop(0, HALF // 2, chunk_body, 0)
        pltpu.make_async_copy(table.at[src_v.at[0]], rows_a, sem_a).wait()
    plsc.subcore_barrier()

    @pl.when(s < 15)
    def _flush_main():
        pltpu.sync_copy(
            acc.at[pl.ds(row0, OWN)],
            out.at[c, pl.ds(row0, OWN)],
        )

    @pl.when(s == 15)
    def _flush_last():
        pltpu.sync_copy(
            acc.at[pl.ds(15 * OWN, OWN_LAST)],
            out.at[c, pl.ds(15 * OWN, OWN_LAST)],
        )


def _aggregate(table, src2d, dst2d):
    mesh = plsc.VectorSubcoreMesh(core_axis_name="c", subcore_axis_name="s")
    kern = functools.partial(
        pl.kernel,
        mesh=mesh,
        out_type=jax.ShapeDtypeStruct((2, N, F), _f32),
        scratch_types=[
            pltpu.VMEM((HALF, CHUNK), jnp.int32),
            pltpu.VMEM((HALF, CHUNK), jnp.int32),
            pltpu.VMEM((CHUNK, F), _f32),
            pltpu.VMEM((CHUNK, F), _f32),
            pltpu.VMEM((16, F), _f32),
            pltpu.VMEM_SHARED((N, F), _f32),
            pltpu.SemaphoreType.DMA,
            pltpu.SemaphoreType.DMA,
        ],
    )(_agg_body)
    return kern(table, src2d, dst2d)


def _count_body(dst2d, out_cnt, dst_v, ones_v, zbuf, cacc):
    c = lax.axis_index("c")
    s = lax.axis_index("s")
    wid = c * 16 + s
    row0 = s * ROWS_PER_TILE

    _fill_zbuf(zbuf, CW)
    _zero_rows(zbuf, cacc, row0, ROWS_PER_TILE // 16, 1)
    onehot = jnp.where(lax.iota(jnp.int32, 16) == 0, 1.0, 0.0).astype(_f32)
    for i in range(CHUNK):
        ones_v[i, pl.ds(0, CW)] = onehot
    plsc.subcore_barrier()

    pltpu.sync_copy(dst2d.at[pl.ds(wid * CPW, CPW)], dst_v)

    def chunk_body(j, carry):
        pltpu.sync_copy(ones_v, cacc.at[dst_v.at[j]], add=True)
        return carry

    lax.fori_loop(0, CPW, chunk_body, 0)
    plsc.subcore_barrier()

    pltpu.sync_copy(
        cacc.at[pl.ds(row0, ROWS_PER_TILE)],
        out_cnt.at[c, pl.ds(row0, ROWS_PER_TILE)],
    )


def _count(dst2d):
    mesh = plsc.VectorSubcoreMesh(core_axis_name="c", subcore_axis_name="s")
    kern = functools.partial(
        pl.kernel,
        mesh=mesh,
        out_type=jax.ShapeDtypeStruct((2, N, CW), _f32),
        scratch_types=[
            pltpu.VMEM((CPW, CHUNK), jnp.int32),
            pltpu.VMEM((CHUNK, CW), _f32),
            pltpu.VMEM((16, CW), _f32),
            pltpu.VMEM_SHARED((N, CW), _f32),
        ],
        compiler_params=pltpu.CompilerParams(use_tc_tiling_on_sc=False),
    )(_count_body)
    return kern(dst2d)


def kernel(x, edge_index, W_enc, b_enc, Wx1, bx1, Wx2, bx2, Wh1, bh1, Wh2, bh2):
    srcp = edge_index[0].reshape(-1, CHUNK)
    dstp = edge_index[1].reshape(-1, CHUNK)

    cnts = _count(dstp)
    parts1 = _aggregate(x, srcp, dstp)
    h_full, x_hat = _finish_enc(
        parts1, cnts, W_enc, b_enc.reshape(1, F),
        Wx1, bx1.reshape(1, F), Wx2, bx2.reshape(1, F),
    )
    parts2 = _aggregate(h_full, srcp, dstp)
    m_hat = _finish_mlp(
        parts2, cnts, Wh1, bh1.reshape(1, F), Wh2, bh2.reshape(1, F),
    )

    return (h_full, x_hat, m_hat)


# speedup vs baseline: 11.7157x; 1.0038x over previous
import functools

import jax
import jax.numpy as jnp
from jax import lax
from jax.experimental import pallas as pl
from jax.experimental.pallas import tpu as pltpu
from jax.experimental.pallas import tpu_sc as plsc

N = 10000
E = 320000
F = 128
CW = 16
CHUNK = 125
CPW = E // 32 // CHUNK
HALF = CPW // 2
ROWS_PER_TILE = N // 16
OWN = 632
OWN_LAST = N - 15 * OWN

_f32 = jnp.float32


def _finish_h_body(parts_ref, cnts_ref, wenc_ref, benc_ref, h_ref):
    tot = parts_ref[0] + parts_ref[1]
    cnt = (cnts_ref[0] + cnts_ref[1])[:, 0:1]
    xm = tot * (1.0 / jnp.maximum(cnt, 1.0))
    h = jnp.dot(xm, wenc_ref[...], preferred_element_type=_f32)
    h_ref[...] = jnp.maximum(h + benc_ref[...], 0.0)


def _finish_h(parts, cnts, wenc, benc, blk=1000):
    grid = N // blk
    mat = pl.BlockSpec((F, F), lambda i: (0, 0))
    vec = pl.BlockSpec((1, F), lambda i: (0, 0))
    return pl.pallas_call(
        _finish_h_body,
        grid=(grid,),
        in_specs=[
            pl.BlockSpec((2, blk, F), lambda i: (0, i, 0)),
            pl.BlockSpec((2, blk, CW), lambda i: (0, i, 0)),
            mat, vec,
        ],
        out_specs=pl.BlockSpec((blk, F), lambda i: (i, 0)),
        out_shape=jax.ShapeDtypeStruct((N, F), _f32),
    )(parts, cnts, wenc, benc)


def _mlp_body(in_ref, w1_ref, b1_ref, w2_ref, b2_ref, out_ref):
    t = jnp.dot(in_ref[...], w1_ref[...], preferred_element_type=_f32)
    t = jnp.maximum(t + b1_ref[...], 0.0)
    out_ref[...] = (
        jnp.dot(t, w2_ref[...], preferred_element_type=_f32) + b2_ref[...]
    )


def _mlp(inp, w1, b1, w2, b2, blk=1000):
    grid = N // blk
    mat = pl.BlockSpec((F, F), lambda i: (0, 0))
    vec = pl.BlockSpec((1, F), lambda i: (0, 0))
    return pl.pallas_call(
        _mlp_body,
        grid=(grid,),
        in_specs=[
            pl.BlockSpec((blk, F), lambda i: (i, 0)),
            mat, vec, mat, vec,
        ],
        out_specs=pl.BlockSpec((blk, F), lambda i: (i, 0)),
        out_shape=jax.ShapeDtypeStruct((N, F), _f32),
    )(inp, w1, b1, w2, b2)


def _finish_mlp_body(parts_ref, cnts_ref, w1_ref, b1_ref, w2_ref, b2_ref,
                     out_ref):
    tot = parts_ref[0] + parts_ref[1]
    cnt = (cnts_ref[0] + cnts_ref[1])[:, 0:1]
    hb = tot * (1.0 / jnp.maximum(cnt, 1.0))
    t = jnp.dot(hb, w1_ref[...], preferred_element_type=_f32)
    t = jnp.maximum(t + b1_ref[...], 0.0)
    out_ref[...] = (
        jnp.dot(t, w2_ref[...], preferred_element_type=_f32) + b2_ref[...]
    )


def _finish_mlp(parts, cnts, w1, b1, w2, b2, blk=1000):
    grid = N // blk
    mat = pl.BlockSpec((F, F), lambda i: (0, 0))
    vec = pl.BlockSpec((1, F), lambda i: (0, 0))
    return pl.pallas_call(
        _finish_mlp_body,
        grid=(grid,),
        in_specs=[
            pl.BlockSpec((2, blk, F), lambda i: (0, i, 0)),
            pl.BlockSpec((2, blk, CW), lambda i: (0, i, 0)),
            mat, vec, mat, vec,
        ],
        out_specs=pl.BlockSpec((blk, F), lambda i: (i, 0)),
        out_shape=jax.ShapeDtypeStruct((N, F), _f32),
    )(parts, cnts, w1, b1, w2, b2)


def _fill_zbuf(zbuf, width):
    z = jnp.zeros((16,), _f32)
    for i in range(16):
        for k in range(width // 16):
            zbuf[i, pl.ds(k * 16, 16)] = z


def _zero_rows(zbuf, acc, row0, nrows16, tail):
    def zero_body(i, carry):
        pltpu.sync_copy(zbuf, acc.at[pl.ds(row0 + i * 16, 16)])
        return carry

    lax.fori_loop(0, nrows16, zero_body, 0)
    if tail:
        pltpu.sync_copy(
            zbuf.at[pl.ds(0, tail)],
            acc.at[pl.ds(row0 + nrows16 * 16, tail)],
        )


def _agg_body(table, src2d, dst2d, out, src_v, dst_v, rows_a, rows_b, zbuf,
              acc, sem_a, sem_b):
    c = lax.axis_index("c")
    s = lax.axis_index("s")
    wid = c * 16 + s
    row0 = s * OWN
    _fill_zbuf(zbuf, F)
    nz16 = jnp.where(s < 15, OWN // 16, OWN_LAST // 16)
    _zero_rows(zbuf, acc, row0, nz16, 8)
    plsc.subcore_barrier()

    for half in range(2):
        base = wid * CPW + half * HALF
        pltpu.sync_copy(src2d.at[pl.ds(base, HALF)], src_v)
        pltpu.sync_copy(dst2d.at[pl.ds(base, HALF)], dst_v)
        pltpu.async_copy(table.at[src_v.at[0]], rows_a, sem_a)

        def chunk_body(j, carry):
            jj = 2 * j
            pltpu.async_copy(table.at[src_v.at[jj + 1]], rows_b, sem_b)
            pltpu.make_async_copy(table.at[src_v.at[jj]], rows_a, sem_a).wait()
            pltpu.sync_copy(rows_a, acc.at[dst_v.at[jj]], add=True)
            nxt = jnp.minimum(jj + 2, HALF - 1)
            pltpu.async_copy(table.at[src_v.at[nxt]], rows_a, sem_a)
            pltpu.make_async_copy(table.at[src_v.at[jj]], rows_b, sem_b).wait()
            pltpu.sync_copy(rows_b, acc.at[dst_v.at[jj + 1]], add=True)
            return carry

        lax.fori_loop(0, HALF // 2, chunk_body, 0)
        pltpu.make_async_copy(table.at[src_v.at[0]], rows_a, sem_a).wait()
    plsc.subcore_barrier()

    @pl.when(s < 15)
    def _flush_main():
        pltpu.sync_copy(
            acc.at[pl.ds(row0, OWN)],
            out.at[c, pl.ds(row0, OWN)],
        )

    @pl.when(s == 15)
    def _flush_last():
        pltpu.sync_copy(
            acc.at[pl.ds(15 * OWN, OWN_LAST)],
            out.at[c, pl.ds(15 * OWN, OWN_LAST)],
        )


def _aggregate(table, src2d, dst2d):
    mesh = plsc.VectorSubcoreMesh(core_axis_name="c", subcore_axis_name="s")
    kern = functools.partial(
        pl.kernel,
        mesh=mesh,
        out_type=jax.ShapeDtypeStruct((2, N, F), _f32),
        scratch_types=[
            pltpu.VMEM((HALF, CHUNK), jnp.int32),
            pltpu.VMEM((HALF, CHUNK), jnp.int32),
            pltpu.VMEM((CHUNK, F), _f32),
            pltpu.VMEM((CHUNK, F), _f32),
            pltpu.VMEM((16, F), _f32),
            pltpu.VMEM_SHARED((N, F), _f32),
            pltpu.SemaphoreType.DMA,
            pltpu.SemaphoreType.DMA,
        ],
    )(_agg_body)
    return kern(table, src2d, dst2d)


def _count_body(dst2d, out_cnt, dst_v, ones_v, zbuf, cacc):
    c = lax.axis_index("c")
    s = lax.axis_index("s")
    wid = c * 16 + s
    row0 = s * ROWS_PER_TILE

    _fill_zbuf(zbuf, CW)
    _zero_rows(zbuf, cacc, row0, ROWS_PER_TILE // 16, 1)
    onehot = jnp.where(lax.iota(jnp.int32, 16) == 0, 1.0, 0.0).astype(_f32)
    for i in range(CHUNK):
        ones_v[i, pl.ds(0, CW)] = onehot
    plsc.subcore_barrier()

    pltpu.sync_copy(dst2d.at[pl.ds(wid * CPW, CPW)], dst_v)

    def chunk_body(j, carry):
        pltpu.sync_copy(ones_v, cacc.at[dst_v.at[j]], add=True)
        return carry

    lax.fori_loop(0, CPW, chunk_body, 0)
    plsc.subcore_barrier()

    pltpu.sync_copy(
        cacc.at[pl.ds(row0, ROWS_PER_TILE)],
        out_cnt.at[c, pl.ds(row0, ROWS_PER_TILE)],
    )


def _count(dst2d):
    mesh = plsc.VectorSubcoreMesh(core_axis_name="c", subcore_axis_name="s")
    kern = functools.partial(
        pl.kernel,
        mesh=mesh,
        out_type=jax.ShapeDtypeStruct((2, N, CW), _f32),
        scratch_types=[
            pltpu.VMEM((CPW, CHUNK), jnp.int32),
            pltpu.VMEM((CHUNK, CW), _f32),
            pltpu.VMEM((16, CW), _f32),
            pltpu.VMEM_SHARED((N, CW), _f32),
        ],
        compiler_params=pltpu.CompilerParams(use_tc_tiling_on_sc=False),
    )(_count_body)
    return kern(dst2d)


def kernel(x, edge_index, W_enc, b_enc, Wx1, bx1, Wx2, bx2, Wh1, bh1, Wh2, bh2):
    srcp = edge_index[0].reshape(-1, CHUNK)
    dstp = edge_index[1].reshape(-1, CHUNK)

    cnts = _count(dstp)
    parts1 = _aggregate(x, srcp, dstp)
    h_full = _finish_h(parts1, cnts, W_enc, b_enc.reshape(1, F))
    parts2 = _aggregate(h_full, srcp, dstp)
    x_hat = _mlp(h_full, Wx1, bx1.reshape(1, F), Wx2, bx2.reshape(1, F))
    m_hat = _finish_mlp(
        parts2, cnts, Wh1, bh1.reshape(1, F), Wh2, bh2.reshape(1, F),
    )

    return (h_full, x_hat, m_hat)


# speedup vs baseline: 11.7158x; 1.0000x over previous
import functools

import jax
import jax.numpy as jnp
from jax import lax
from jax.experimental import pallas as pl
from jax.experimental.pallas import tpu as pltpu
from jax.experimental.pallas import tpu_sc as plsc

N = 10000
E = 320000
F = 128
CW = 16
CHUNK = 128
EPAD = 2560 * CHUNK
CPW = 2560 // 32
HALF = CPW // 2
NACC = 10240
ROWS_PER_TILE = NACC // 16
CNT_PER_TILE = N // 16
OWN = 632
OWN_LAST = N - 15 * OWN

_f32 = jnp.float32


def _finish_h_body(parts_ref, cnts_ref, wenc_ref, benc_ref, h_ref):
    tot = parts_ref[0] + parts_ref[1]
    cnt = (cnts_ref[0] + cnts_ref[1])[:, 0:1]
    xm = tot * (1.0 / jnp.maximum(cnt, 1.0))
    h = jnp.dot(xm, wenc_ref[...], preferred_element_type=_f32)
    h_ref[...] = jnp.maximum(h + benc_ref[...], 0.0)


def _finish_h(parts, cnts, wenc, benc, blk=1000):
    grid = N // blk
    mat = pl.BlockSpec((F, F), lambda i: (0, 0))
    vec = pl.BlockSpec((1, F), lambda i: (0, 0))
    return pl.pallas_call(
        _finish_h_body,
        grid=(grid,),
        in_specs=[
            pl.BlockSpec((2, blk, F), lambda i: (0, i, 0)),
            pl.BlockSpec((2, blk, CW), lambda i: (0, i, 0)),
            mat, vec,
        ],
        out_specs=pl.BlockSpec((blk, F), lambda i: (i, 0)),
        out_shape=jax.ShapeDtypeStruct((N, F), _f32),
    )(parts, cnts, wenc, benc)


def _mlp_body(in_ref, w1_ref, b1_ref, w2_ref, b2_ref, out_ref):
    t = jnp.dot(in_ref[...], w1_ref[...], preferred_element_type=_f32)
    t = jnp.maximum(t + b1_ref[...], 0.0)
    out_ref[...] = (
        jnp.dot(t, w2_ref[...], preferred_element_type=_f32) + b2_ref[...]
    )


def _mlp(inp, w1, b1, w2, b2, blk=1000):
    grid = N // blk
    mat = pl.BlockSpec((F, F), lambda i: (0, 0))
    vec = pl.BlockSpec((1, F), lambda i: (0, 0))
    return pl.pallas_call(
        _mlp_body,
        grid=(grid,),
        in_specs=[
            pl.BlockSpec((blk, F), lambda i: (i, 0)),
            mat, vec, mat, vec,
        ],
        out_specs=pl.BlockSpec((blk, F), lambda i: (i, 0)),
        out_shape=jax.ShapeDtypeStruct((N, F), _f32),
    )(inp, w1, b1, w2, b2)


def _finish_mlp_body(parts_ref, cnts_ref, w1_ref, b1_ref, w2_ref, b2_ref,
                     out_ref):
    tot = parts_ref[0] + parts_ref[1]
    cnt = (cnts_ref[0] + cnts_ref[1])[:, 0:1]
    hb = tot * (1.0 / jnp.maximum(cnt, 1.0))
    t = jnp.dot(hb, w1_ref[...], preferred_element_type=_f32)
    t = jnp.maximum(t + b1_ref[...], 0.0)
    out_ref[...] = (
        jnp.dot(t, w2_ref[...], preferred_element_type=_f32) + b2_ref[...]
    )


def _finish_mlp(parts, cnts, w1, b1, w2, b2, blk=1000):
    grid = N // blk
    mat = pl.BlockSpec((F, F), lambda i: (0, 0))
    vec = pl.BlockSpec((1, F), lambda i: (0, 0))
    return pl.pallas_call(
        _finish_mlp_body,
        grid=(grid,),
        in_specs=[
            pl.BlockSpec((2, blk, F), lambda i: (0, i, 0)),
            pl.BlockSpec((2, blk, CW), lambda i: (0, i, 0)),
            mat, vec, mat, vec,
        ],
        out_specs=pl.BlockSpec((blk, F), lambda i: (i, 0)),
        out_shape=jax.ShapeDtypeStruct((N, F), _f32),
    )(parts, cnts, w1, b1, w2, b2)


def _fill_zbuf(zbuf, width):
    z = jnp.zeros((16,), _f32)
    for i in range(16):
        for k in range(width // 16):
            zbuf[i, pl.ds(k * 16, 16)] = z


def _zero_rows(zbuf, acc, row0, nrows16, tail):
    def zero_body(i, carry):
        pltpu.sync_copy(zbuf, acc.at[pl.ds(row0 + i * 16, 16)])
        return carry

    lax.fori_loop(0, nrows16, zero_body, 0)
    if tail:
        pltpu.sync_copy(
            zbuf.at[pl.ds(0, tail)],
            acc.at[pl.ds(row0 + nrows16 * 16, tail)],
        )


def _agg_body(table, src2d, dst2d, out, src_v, dst_v, rows_a, rows_b, zbuf,
              acc, sem_a, sem_b):
    c = lax.axis_index("c")
    s = lax.axis_index("s")
    wid = c * 16 + s
    _fill_zbuf(zbuf, F)
    _zero_rows(zbuf, acc, s * ROWS_PER_TILE, ROWS_PER_TILE // 16, 0)
    plsc.subcore_barrier()

    for half in range(2):
        base = wid * CPW + half * HALF
        pltpu.sync_copy(src2d.at[pl.ds(base, HALF)], src_v)
        pltpu.sync_copy(dst2d.at[pl.ds(base, HALF)], dst_v)
        pltpu.async_copy(table.at[src_v.at[0]], rows_a, sem_a)

        def chunk_body(j, carry):
            jj = 2 * j
            pltpu.async_copy(table.at[src_v.at[jj + 1]], rows_b, sem_b)
            pltpu.make_async_copy(table.at[src_v.at[jj]], rows_a, sem_a).wait()
            pltpu.sync_copy(rows_a, acc.at[dst_v.at[jj]], add=True)
            nxt = jnp.minimum(jj + 2, HALF - 1)
            pltpu.async_copy(table.at[src_v.at[nxt]], rows_a, sem_a)
            pltpu.make_async_copy(table.at[src_v.at[jj]], rows_b, sem_b).wait()
            pltpu.sync_copy(rows_b, acc.at[dst_v.at[jj + 1]], add=True)
            return carry

        lax.fori_loop(0, HALF // 2, chunk_body, 0)
        pltpu.make_async_copy(table.at[src_v.at[0]], rows_a, sem_a).wait()
    plsc.subcore_barrier()

    @pl.when(s < 15)
    def _flush_main():
        pltpu.sync_copy(
            acc.at[pl.ds(s * OWN, OWN)],
            out.at[c, pl.ds(s * OWN, OWN)],
        )

    @pl.when(s == 15)
    def _flush_last():
        pltpu.sync_copy(
            acc.at[pl.ds(15 * OWN, OWN_LAST)],
            out.at[c, pl.ds(15 * OWN, OWN_LAST)],
        )


def _aggregate(table, src2d, dst2d):
    mesh = plsc.VectorSubcoreMesh(core_axis_name="c", subcore_axis_name="s")
    kern = functools.partial(
        pl.kernel,
        mesh=mesh,
        out_type=jax.ShapeDtypeStruct((2, N, F), _f32),
        scratch_types=[
            pltpu.VMEM((HALF, CHUNK), jnp.int32),
            pltpu.VMEM((HALF, CHUNK), jnp.int32),
            pltpu.VMEM((CHUNK, F), _f32),
            pltpu.VMEM((CHUNK, F), _f32),
            pltpu.VMEM((16, F), _f32),
            pltpu.VMEM_SHARED((NACC, F), _f32),
            pltpu.SemaphoreType.DMA,
            pltpu.SemaphoreType.DMA,
        ],
    )(_agg_body)
    return kern(table, src2d, dst2d)


def _count_body(dst2d, out_cnt, dst_v, ones_v, zbuf, cacc):
    c = lax.axis_index("c")
    s = lax.axis_index("s")
    wid = c * 16 + s
    _fill_zbuf(zbuf, CW)
    _zero_rows(zbuf, cacc, s * ROWS_PER_TILE, ROWS_PER_TILE // 16, 0)
    onehot = jnp.where(lax.iota(jnp.int32, 16) == 0, 1.0, 0.0).astype(_f32)
    for i in range(CHUNK):
        ones_v[i, pl.ds(0, CW)] = onehot
    plsc.subcore_barrier()

    pltpu.sync_copy(dst2d.at[pl.ds(wid * CPW, CPW)], dst_v)

    def chunk_body(j, carry):
        pltpu.sync_copy(ones_v, cacc.at[dst_v.at[j]], add=True)
        return carry

    lax.fori_loop(0, CPW, chunk_body, 0)
    plsc.subcore_barrier()

    pltpu.sync_copy(
        cacc.at[pl.ds(s * CNT_PER_TILE, CNT_PER_TILE)],
        out_cnt.at[c, pl.ds(s * CNT_PER_TILE, CNT_PER_TILE)],
    )


def _count(dst2d):
    mesh = plsc.VectorSubcoreMesh(core_axis_name="c", subcore_axis_name="s")
    kern = functools.partial(
        pl.kernel,
        mesh=mesh,
        out_type=jax.ShapeDtypeStruct((2, N, CW), _f32),
        scratch_types=[
            pltpu.VMEM((CPW, CHUNK), jnp.int32),
            pltpu.VMEM((CHUNK, CW), _f32),
            pltpu.VMEM((16, CW), _f32),
            pltpu.VMEM_SHARED((NACC, CW), _f32),
        ],
        compiler_params=pltpu.CompilerParams(use_tc_tiling_on_sc=False),
    )(_count_body)
    return kern(dst2d)


def kernel(x, edge_index, W_enc, b_enc, Wx1, bx1, Wx2, bx2, Wh1, bh1, Wh2, bh2):
    pad = EPAD - E
    pad_ids = lax.iota(jnp.int32, pad)
    srcp = jnp.concatenate([edge_index[0], pad_ids % N]).reshape(-1, CHUNK)
    dstp = jnp.concatenate(
        [edge_index[1], N + pad_ids % (NACC - N)]).reshape(-1, CHUNK)

    cnts = _count(dstp)
    parts1 = _aggregate(x, srcp, dstp)
    h_full = _finish_h(parts1, cnts, W_enc, b_enc.reshape(1, F))
    parts2 = _aggregate(h_full, srcp, dstp)
    x_hat = _mlp(h_full, Wx1, bx1.reshape(1, F), Wx2, bx2.reshape(1, F))
    m_hat = _finish_mlp(
        parts2, cnts, Wh1, bh1.reshape(1, F), Wh2, bh2.reshape(1, F),
    )

    return (h_full, x_hat, m_hat)
